# baseline (speedup 1.0000x reference)
import functools

import jax
import jax.numpy as jnp
from jax import lax
from jax.experimental import pallas as pl
from jax.experimental.pallas import tpu as pltpu
from jax.experimental.pallas import tpu_sc as plsc

B = 16384
NN_DIM = 64
MF_DIM = 32

_NC = 2
_NS = 16
_NW = _NC * _NS
_BPW = B // _NW
_CHUNK = 128
_NCHUNK = _BPW // _CHUNK

_BLK = 2048
_NBLK = B // _BLK


def _gather_body(user_hbm, item_hbm, nn_u_hbm, nn_i_hbm, mf_u_hbm, mf_i_hbm,
                 out_nn_u, out_nn_i, out_mf_u, out_mf_i,
                 uidx_v, iidx_v, nnu_v, nni_v, mfu_v, mfi_v, sem):
    wid = lax.axis_index("s") * _NC + lax.axis_index("c")
    base = wid * _BPW
    pltpu.sync_copy(user_hbm.at[pl.ds(base, _BPW)], uidx_v)
    pltpu.sync_copy(item_hbm.at[pl.ds(base, _BPW)], iidx_v)
    copies = []
    for j in range(_NCHUNK):
        sl = pl.ds(j * _CHUNK, _CHUNK)
        copies.append(pltpu.async_copy(nn_u_hbm.at[uidx_v.at[sl]], nnu_v.at[sl, :], sem))
        copies.append(pltpu.async_copy(nn_i_hbm.at[iidx_v.at[sl]], nni_v.at[sl, :], sem))
        copies.append(pltpu.async_copy(mf_u_hbm.at[uidx_v.at[sl]], mfu_v.at[sl, :], sem))
        copies.append(pltpu.async_copy(mf_i_hbm.at[iidx_v.at[sl]], mfi_v.at[sl, :], sem))
    for c in copies:
        c.wait()
    out_sl = pl.ds(base, _BPW)
    pltpu.sync_copy(nnu_v, out_nn_u.at[out_sl, :])
    pltpu.sync_copy(nni_v, out_nn_i.at[out_sl, :])
    pltpu.sync_copy(mfu_v, out_mf_u.at[out_sl, :])
    pltpu.sync_copy(mfi_v, out_mf_i.at[out_sl, :])


def _sc_gather(user, item, nn_usr, nn_item, mf_usr, mf_item):
    mesh = plsc.VectorSubcoreMesh(core_axis_name="c", subcore_axis_name="s")
    f32 = jnp.float32
    return pl.kernel(
        _gather_body,
        out_type=[
            jax.ShapeDtypeStruct((B, NN_DIM), f32),
            jax.ShapeDtypeStruct((B, NN_DIM), f32),
            jax.ShapeDtypeStruct((B, MF_DIM), f32),
            jax.ShapeDtypeStruct((B, MF_DIM), f32),
        ],
        mesh=mesh,
        scratch_types=[
            pltpu.VMEM((_BPW,), jnp.int32),
            pltpu.VMEM((_BPW,), jnp.int32),
            pltpu.VMEM((_BPW, NN_DIM), f32),
            pltpu.VMEM((_BPW, NN_DIM), f32),
            pltpu.VMEM((_BPW, MF_DIM), f32),
            pltpu.VMEM((_BPW, MF_DIM), f32),
            pltpu.SemaphoreType.DMA,
        ],
        compiler_params=pltpu.CompilerParams(use_tc_tiling_on_sc=False),
    )(user, item, nn_usr, nn_item, mf_usr, mf_item)


def _mlp_body(nnu, nni, mfu, mfi, w1u, w1i, b1, w2, b2, w3, b3, wmf, wx, bo, out):
    hp = lax.Precision.HIGHEST
    f32 = jnp.float32
    x = jnp.dot(nnu[...], w1u[...], precision=hp, preferred_element_type=f32)
    x = x + jnp.dot(nni[...], w1i[...], precision=hp, preferred_element_type=f32)
    x = jnp.maximum(x + b1[...], 0.0)
    x = jnp.maximum(jnp.dot(x, w2[...], precision=hp, preferred_element_type=f32) + b2[...], 0.0)
    x = jnp.maximum(jnp.dot(x, w3[...], precision=hp, preferred_element_type=f32) + b3[...], 0.0)
    mf = mfu[...] * mfi[...]
    acc = jnp.sum(mf * wmf[...], axis=1) + jnp.sum(x * wx[...], axis=1) + bo[0, 0]
    out[0, 0, :] = acc


def kernel(user, item, mf_usr, mf_item, nn_usr, nn_item,
           fc1_w, fc1_b, fc2_w, fc2_b, fc3_w, fc3_b, neumf_w, neumf_b):
    user = user.astype(jnp.int32)
    item = item.astype(jnp.int32)
    nn_u, nn_i, mf_u, mf_i = _sc_gather(user, item, nn_usr, nn_item, mf_usr, mf_item)

    w1 = fc1_w.T
    w1u, w1i = w1[:NN_DIM], w1[NN_DIM:]
    w2 = fc2_w.T
    w3 = fc3_w.T
    wmf = neumf_w[:, :MF_DIM]
    wx = neumf_w[:, MF_DIM:]

    full = lambda shape: pl.BlockSpec(shape, lambda i: (0, 0))
    row = lambda d: pl.BlockSpec((_BLK, d), lambda i: (i, 0))
    out2d = pl.pallas_call(
        _mlp_body,
        grid=(_NBLK,),
        in_specs=[
            row(NN_DIM), row(NN_DIM), row(MF_DIM), row(MF_DIM),
            full((NN_DIM, 128)), full((NN_DIM, 128)), full((1, 128)),
            full((128, 64)), full((1, 64)),
            full((64, 32)), full((1, 32)),
            full((1, 32)), full((1, 32)), full((1, 1)),
        ],
        out_specs=pl.BlockSpec((1, 1, _BLK), lambda i: (i, 0, 0)),
        out_shape=jax.ShapeDtypeStruct((_NBLK, 1, _BLK), jnp.float32),
    )(nn_u, nn_i, mf_u, mf_i, w1u, w1i, fc1_b[None], w2, fc2_b[None],
      w3, fc3_b[None], wmf, wx, neumf_b[None])
    return out2d.reshape(B)
